# baseline (speedup 1.0000x reference)
import functools

import jax
import jax.numpy as jnp
from jax import lax
from jax.experimental import pallas as pl
from jax.experimental.pallas import tpu as pltpu
from jax.experimental.pallas import tpu_sc as plsc

E = 16
TOPK = 2
D = 1024
FF = 2048
T = 128

FFB = 1024
NFF = FF // FFB

_NC = 2
_NS = 16
_NW = _NC * _NS
_TPW = T // _NW


def _route_sc_body(ltr_hbm, gout_hbm, lbuf, gbuf, sem):
    wid = lax.axis_index("s") * _NC + lax.axis_index("c")

    @pl.when(wid < T // 16)
    def _():
        base = wid * 16
        pltpu.sync_copy(ltr_hbm, lbuf)
        cols = [lbuf[e, pl.ds(base, 16)] for e in range(E)]
        m1 = cols[0]
        i1 = jnp.zeros((16,), jnp.int32)
        for e in range(1, E):
            better = cols[e] > m1
            m1 = jnp.where(better, cols[e], m1)
            i1 = jnp.where(better, e, i1)
        neg = jnp.full((16,), -jnp.inf, jnp.float32)
        m2 = jnp.where(i1 == 0, neg, cols[0])
        i2 = jnp.zeros((16,), jnp.int32)
        for e in range(1, E):
            ce = jnp.where(i1 == e, neg, cols[e])
            better = ce > m2
            m2 = jnp.where(better, ce, m2)
            i2 = jnp.where(better, e, i2)
        ga = 1.0 / (1.0 + jnp.exp(m2 - m1))
        zero = jnp.zeros((16,), jnp.float32)
        for e in range(E):
            gbuf[e, :] = (jnp.where(i1 == e, ga, zero)
                          + jnp.where(i2 == e, 1.0 - ga, zero))
        pltpu.sync_copy(gbuf, gout_hbm.at[wid])


def _route_sc(router_logits):
    mesh = plsc.VectorSubcoreMesh(core_axis_name="c", subcore_axis_name="s")
    fn = pl.kernel(
        _route_sc_body,
        mesh=mesh,
        out_type=jax.ShapeDtypeStruct((T // 16, E, 16), jnp.float32),
        scratch_types=[
            pltpu.VMEM((E, T), jnp.float32),
            pltpu.VMEM((E, 16), jnp.float32),
            pltpu.SemaphoreType.DMA,
        ],
    )
    gout = fn(router_logits.T)
    return gout.transpose(1, 0, 2).reshape(E, T)


def _moe_body(gate_ref, x_ref, w1_ref, w3_ref, w2_ref, out_ref):
    e = pl.program_id(0)
    ff = pl.program_id(1)

    @pl.when((e == 0) & (ff == 0))
    def _():
        out_ref[...] = jnp.zeros_like(out_ref)

    x = x_ref[...]
    dn = (((1,), (1,)), ((), ()))
    g = lax.dot_general(x, w1_ref[0], dn, preferred_element_type=jnp.float32)
    u = lax.dot_general(x, w3_ref[0], dn, preferred_element_type=jnp.float32)
    act = g * (1.0 / (1.0 + jnp.exp(-g))) * u
    oh = (lax.broadcasted_iota(jnp.int32, (E, 1), 0) == e).astype(jnp.float32)
    gcol = lax.dot_general(gate_ref[...], oh, (((0,), (0,)), ((), ())),
                           preferred_element_type=jnp.float32)
    act = act * gcol
    out_ref[...] += lax.dot_general(act, w2_ref[0], dn,
                                    preferred_element_type=jnp.float32)


def kernel(hidden_states, router_logits, w13, w2):
    gate = _route_sc(router_logits)
    return pl.pallas_call(
        _moe_body,
        grid=(E, NFF),
        in_specs=[
            pl.BlockSpec((E, T), lambda e, ff: (0, 0)),
            pl.BlockSpec((T, D), lambda e, ff: (0, 0)),
            pl.BlockSpec((1, FFB, D), lambda e, ff: (e, ff, 0)),
            pl.BlockSpec((1, FFB, D), lambda e, ff: (e, NFF + ff, 0)),
            pl.BlockSpec((1, D, FFB), lambda e, ff: (e, 0, ff)),
        ],
        out_specs=pl.BlockSpec((T, D), lambda e, ff: (0, 0)),
        out_shape=jax.ShapeDtypeStruct((T, D), jnp.float32),
        compiler_params=pltpu.CompilerParams(
            dimension_semantics=("arbitrary", "arbitrary")),
    )(gate, hidden_states, w13, w13, w2)


# speedup vs baseline: 1.0093x; 1.0093x over previous
import jax
import jax.numpy as jnp
from jax import lax
from jax.experimental import pallas as pl
from jax.experimental.pallas import tpu as pltpu
from jax.experimental.pallas import tpu_sc as plsc

E = 16
TOPK = 2
D = 1024
FF = 2048
T = 128
NB = T // 16

FFB = 1024
NFF = FF // FFB

_NC = 2
_NS = 16


def _route_sc_body(ltr_hbm, gout_hbm, lbuf, gbuf, sem):
    wid = lax.axis_index("s") * _NC + lax.axis_index("c")

    @pl.when(wid < NB)
    def _():
        base = wid * 16
        pltpu.sync_copy(ltr_hbm, lbuf)
        cols = [lbuf[e, pl.ds(base, 16)] for e in range(E)]
        m1 = cols[0]
        i1 = jnp.zeros((16,), jnp.int32)
        for e in range(1, E):
            better = cols[e] > m1
            m1 = jnp.where(better, cols[e], m1)
            i1 = jnp.where(better, e, i1)
        neg = jnp.full((16,), -jnp.inf, jnp.float32)
        m2 = jnp.where(i1 == 0, neg, cols[0])
        i2 = jnp.zeros((16,), jnp.int32)
        for e in range(1, E):
            ce = jnp.where(i1 == e, neg, cols[e])
            better = ce > m2
            m2 = jnp.where(better, ce, m2)
            i2 = jnp.where(better, e, i2)
        ga = 1.0 / (1.0 + jnp.exp(m2 - m1))
        zero = jnp.zeros((16,), jnp.float32)
        for e in range(E):
            gbuf[e, :] = (jnp.where(i1 == e, ga, zero)
                          + jnp.where(i2 == e, 1.0 - ga, zero))
        pltpu.sync_copy(gbuf, gout_hbm.at[wid])


def _route_sc(router_logits):
    mesh = plsc.VectorSubcoreMesh(core_axis_name="c", subcore_axis_name="s")
    fn = pl.kernel(
        _route_sc_body,
        mesh=mesh,
        out_type=jax.ShapeDtypeStruct((NB, E, 16), jnp.float32),
        scratch_types=[
            pltpu.VMEM((E, T), jnp.float32),
            pltpu.VMEM((E, 16), jnp.float32),
            pltpu.SemaphoreType.DMA,
        ],
    )
    return fn(router_logits.T)


def _moe_body(gate_ref, x_ref, w1_ref, w3_ref, w2_ref, out_ref):
    e = pl.program_id(0)
    ff = pl.program_id(1)

    @pl.when((e == 0) & (ff == 0))
    def _():
        out_ref[...] = jnp.zeros_like(out_ref)

    x = x_ref[...]
    dn = (((1,), (1,)), ((), ()))
    g = lax.dot_general(x, w1_ref[0], dn, preferred_element_type=jnp.float32)
    u = lax.dot_general(x, w3_ref[0], dn, preferred_element_type=jnp.float32)
    act = g * (1.0 / (1.0 + jnp.exp(-g))) * u

    g2 = gate_ref[...].reshape(NB * E, 16)
    trow = lax.broadcasted_iota(jnp.int32, (T, NB * E), 0)
    ccol = lax.broadcasted_iota(jnp.int32, (T, NB * E), 1)
    a1 = (ccol == (trow // 16) * E + e).astype(jnp.float32)
    p = lax.dot_general(a1, g2, (((1,), (0,)), ((), ())),
                        preferred_element_type=jnp.float32)
    ti = lax.broadcasted_iota(jnp.int32, (T, 16), 0)
    li = lax.broadcasted_iota(jnp.int32, (T, 16), 1)
    gcol = jnp.sum(jnp.where(li == ti % 16, p, 0.0), axis=-1, keepdims=True)

    act = act * gcol
    out_ref[...] += lax.dot_general(act, w2_ref[0], dn,
                                    preferred_element_type=jnp.float32)


def kernel(hidden_states, router_logits, w13, w2):
    gate = _route_sc(router_logits)
    return pl.pallas_call(
        _moe_body,
        grid=(E, NFF),
        in_specs=[
            pl.BlockSpec((NB, E, 16), lambda e, ff: (0, 0, 0)),
            pl.BlockSpec((T, D), lambda e, ff: (0, 0)),
            pl.BlockSpec((1, FFB, D), lambda e, ff: (e, ff, 0)),
            pl.BlockSpec((1, FFB, D), lambda e, ff: (e, NFF + ff, 0)),
            pl.BlockSpec((1, D, FFB), lambda e, ff: (e, 0, ff)),
        ],
        out_specs=pl.BlockSpec((T, D), lambda e, ff: (0, 0)),
        out_shape=jax.ShapeDtypeStruct((T, D), jnp.float32),
        compiler_params=pltpu.CompilerParams(
            dimension_semantics=("arbitrary", "arbitrary")),
    )(gate, hidden_states, w13, w13, w2)
